# baseline (speedup 1.0000x reference)
import functools

import jax
import jax.numpy as jnp
from jax import lax
from jax.experimental import pallas as pl
from jax.experimental.pallas import tpu as pltpu
from jax.experimental.pallas import tpu_sc as plsc

_N = 10000
_NP = 10240
_E = 320000
_D = 128
_G = 64
_CH = 128
_NC = 2
_NS = 16
_NW = _NC * _NS
_CPW = 79
_TCB = 1024


def _sc_degrees(src3, dst3):
    mesh = plsc.VectorSubcoreMesh(core_axis_name="c", subcore_axis_name="s")

    @functools.partial(
        pl.kernel,
        mesh=mesh,
        out_type=jax.ShapeDtypeStruct((_NC, 2, _NP), jnp.float32),
        scratch_types=[
            pltpu.VMEM((_CPW, _CH), jnp.int32),
            pltpu.VMEM((_CPW, _CH), jnp.int32),
            pltpu.VMEM((_CH,), jnp.float32),
            pltpu.VMEM((1024,), jnp.float32),
            pltpu.VMEM_SHARED((_NP,), jnp.float32),
            pltpu.VMEM_SHARED((_NP,), jnp.float32),
            pltpu.SemaphoreType.DMA,
            pltpu.SemaphoreType.DMA,
        ],
    )
    def deg_kernel(src_hbm, dst_hbm, out_hbm, idxs2, idxd2, ones_v, zb_v,
                   acc_s, acc_d, sem_s, sem_d):
        c = lax.axis_index("c")
        s = lax.axis_index("s")
        wid = s * _NC + c
        pltpu.sync_copy(src_hbm.at[wid], idxs2)
        pltpu.sync_copy(dst_hbm.at[wid], idxd2)
        for j in range(_CH // 16):
            ones_v[pl.ds(j * 16, 16)] = jnp.ones((16,), jnp.float32)

        @pl.when(s == 0)
        def _zero():
            for j in range(1024 // 16):
                zb_v[pl.ds(j * 16, 16)] = jnp.zeros((16,), jnp.float32)
            for k in range(_NP // 1024):
                pltpu.sync_copy(zb_v, acc_s.at[pl.ds(k * 1024, 1024)])
                pltpu.sync_copy(zb_v, acc_d.at[pl.ds(k * 1024, 1024)])

        plsc.subcore_barrier()

        def body(j, carry):
            pltpu.async_copy(ones_v, acc_s.at[idxs2.at[j]], sem_s, add=True)
            pltpu.async_copy(ones_v, acc_d.at[idxd2.at[j]], sem_d, add=True)
            return carry

        lax.fori_loop(0, _CPW, body, 0)

        def drain(j, carry):
            pltpu.make_async_copy(ones_v, acc_s.at[idxs2.at[0]], sem_s).wait()
            pltpu.make_async_copy(ones_v, acc_d.at[idxd2.at[0]], sem_d).wait()
            return carry

        lax.fori_loop(0, _CPW, drain, 0)
        plsc.subcore_barrier()

        @pl.when(s == 0)
        def _out():
            pltpu.sync_copy(acc_s, out_hbm.at[c, 0])
            pltpu.sync_copy(acc_d, out_hbm.at[c, 1])

    return deg_kernel(src3, dst3)


def _sc_conv(h, src3, dst3):
    mesh = plsc.VectorSubcoreMesh(core_axis_name="c", subcore_axis_name="s")

    @functools.partial(
        pl.kernel,
        mesh=mesh,
        out_type=jax.ShapeDtypeStruct((_NC, _NP, _D), jnp.float32),
        scratch_types=[
            pltpu.VMEM((_CPW, _CH), jnp.int32),
            pltpu.VMEM((_CH,), jnp.int32),
            pltpu.VMEM((_CH,), jnp.int32),
            pltpu.VMEM((_CH, _D), jnp.float32),
            pltpu.VMEM((_CH, _D), jnp.float32),
            pltpu.VMEM_SHARED((_NP, _D), jnp.float32),
            pltpu.SemaphoreType.DMA,
            pltpu.SemaphoreType.DMA,
            pltpu.SemaphoreType.DMA,
            pltpu.SemaphoreType.DMA,
            pltpu.SemaphoreType.DMA,
        ],
    )
    def conv_kernel(h_hbm, src_hbm, dst_hbm, out_hbm, idxs2, idxd_a, idxd_b,
                    rows_a, rows_b, acc, sem_a, sem_b, sem_ia, sem_ib, sem_z):
        c = lax.axis_index("c")
        s = lax.axis_index("s")
        wid = s * _NC + c
        slab = pltpu.async_copy(src_hbm.at[wid], idxs2, sem_z)

        def zrow(i, carry):
            for j in range(_D // 16):
                rows_a[i, pl.ds(j * 16, 16)] = jnp.zeros((16,), jnp.float32)
            return carry

        lax.fori_loop(0, _CH, zrow, 0)
        slab.wait()
        for k in range(5):
            pltpu.async_copy(rows_a, acc.at[pl.ds((s + 16 * k) * _CH, _CH)],
                             sem_z)
        for k in range(5):
            pltpu.make_async_copy(rows_a, acc.at[pl.ds(0, _CH)], sem_z).wait()
        plsc.subcore_barrier()

        def wait_a():
            pltpu.make_async_copy(h_hbm.at[pl.ds(0, _CH)], rows_a, sem_a).wait()
            pltpu.make_async_copy(dst_hbm.at[0, 0], idxd_a, sem_ia).wait()

        def wait_b():
            pltpu.make_async_copy(h_hbm.at[pl.ds(0, _CH)], rows_b, sem_b).wait()
            pltpu.make_async_copy(dst_hbm.at[0, 0], idxd_b, sem_ib).wait()

        pltpu.async_copy(h_hbm.at[idxs2.at[0]], rows_a, sem_a)
        pltpu.async_copy(dst_hbm.at[wid, 0], idxd_a, sem_ia)

        def body(i, carry):
            pltpu.async_copy(h_hbm.at[idxs2.at[2 * i + 1]], rows_b, sem_b)
            pltpu.async_copy(dst_hbm.at[wid, 2 * i + 1], idxd_b, sem_ib)
            wait_a()
            pltpu.sync_copy(rows_a, acc.at[idxd_a], add=True)
            pltpu.async_copy(h_hbm.at[idxs2.at[2 * i + 2]], rows_a, sem_a)
            pltpu.async_copy(dst_hbm.at[wid, 2 * i + 2], idxd_a, sem_ia)
            wait_b()
            pltpu.sync_copy(rows_b, acc.at[idxd_b], add=True)
            return carry

        lax.fori_loop(0, (_CPW - 1) // 2, body, 0)
        wait_a()
        pltpu.sync_copy(rows_a, acc.at[idxd_a], add=True)

        plsc.subcore_barrier()
        for k in range(5):
            off = (s + 16 * k) * _CH
            pltpu.async_copy(acc.at[pl.ds(off, _CH)],
                             out_hbm.at[c, pl.ds(off, _CH)], sem_z)
        for k in range(5):
            pltpu.make_async_copy(acc.at[pl.ds(0, _CH)],
                                  out_hbm.at[c, pl.ds(0, _CH)], sem_z).wait()

    return conv_kernel(h, src3, dst3)


def _norms(p, col):
    deg = p[0, col] + p[1, col]
    return jnp.where(deg > 0, lax.rsqrt(deg), 0.0)


def _tc_embed(x, w, b2, parts4):
    def kf(x_ref, w_ref, b_ref, p_ref, o_ref):
        h = jnp.dot(x_ref[...], w_ref[...],
                    preferred_element_type=jnp.float32) + b_ref[...]
        h = h * jax.nn.sigmoid(h)
        o_ref[...] = h * _norms(p_ref[...], 0)

    return pl.pallas_call(
        kf,
        grid=(_NP // _TCB,),
        in_specs=[
            pl.BlockSpec((_TCB, _D), lambda i: (i, 0)),
            pl.BlockSpec((_D, _D), lambda i: (0, 0)),
            pl.BlockSpec((1, _D), lambda i: (0, 0)),
            pl.BlockSpec((2, 2, _TCB, 1), lambda i: (0, 0, i, 0)),
        ],
        out_specs=pl.BlockSpec((_TCB, _D), lambda i: (i, 0)),
        out_shape=jax.ShapeDtypeStruct((_NP, _D), jnp.float32),
    )(x, w, b2, parts4)


def _tc_update(agg, parts4, w, b2):
    def kf(a_ref, p_ref, w_ref, b_ref, o_ref):
        p = p_ref[...]
        a = (a_ref[0] + a_ref[1]) * _norms(p, 1)
        h = jnp.dot(a, w_ref[...], preferred_element_type=jnp.float32) + b_ref[...]
        h = h * jax.nn.sigmoid(h)
        o_ref[...] = h * _norms(p, 0)

    return pl.pallas_call(
        kf,
        grid=(_NP // _TCB,),
        in_specs=[
            pl.BlockSpec((_NC, _TCB, _D), lambda i: (0, i, 0)),
            pl.BlockSpec((2, 2, _TCB, 1), lambda i: (0, 0, i, 0)),
            pl.BlockSpec((_D, _D), lambda i: (0, 0)),
            pl.BlockSpec((1, _D), lambda i: (0, 0)),
        ],
        out_specs=pl.BlockSpec((_TCB, _D), lambda i: (i, 0)),
        out_shape=jax.ShapeDtypeStruct((_NP, _D), jnp.float32),
    )(agg, parts4, w, b2)


def _tc_final(agg, parts4, gids2, w2, b2_2, w_out, b_out2, w_ff, b_ff2):
    def kf(a_ref, p_ref, g_ref, w2_ref, b2_ref, wo_ref, bo_ref, wf_ref, bf_ref,
           o_ref):
        i = pl.program_id(0)
        a = (a_ref[0] + a_ref[1]) * _norms(p_ref[...], 1)
        h = jnp.dot(a, w2_ref[...], preferred_element_type=jnp.float32) + b2_ref[...]
        h = h * jax.nn.sigmoid(h)
        h = jnp.dot(h, wo_ref[...], preferred_element_type=jnp.float32) + bo_ref[...]
        h = h * jax.nn.sigmoid(h)
        hw = jnp.dot(h, wf_ref[...], preferred_element_type=jnp.float32)
        onehot = (g_ref[...] == lax.broadcasted_iota(jnp.int32, (1, _G), 1))
        part = lax.dot_general(onehot.astype(jnp.float32), hw,
                               (((0,), (0,)), ((), ())),
                               preferred_element_type=jnp.float32)

        @pl.when(i == 0)
        def _init():
            o_ref[...] = jnp.zeros_like(o_ref)

        o_ref[...] += part

        @pl.when(i == pl.num_programs(0) - 1)
        def _bias():
            o_ref[...] += bf_ref[...]

    return pl.pallas_call(
        kf,
        grid=(_NP // _TCB,),
        in_specs=[
            pl.BlockSpec((_NC, _TCB, _D), lambda i: (0, i, 0)),
            pl.BlockSpec((2, 2, _TCB, 1), lambda i: (0, 0, i, 0)),
            pl.BlockSpec((_TCB, 1), lambda i: (i, 0)),
            pl.BlockSpec((_D, _D), lambda i: (0, 0)),
            pl.BlockSpec((1, _D), lambda i: (0, 0)),
            pl.BlockSpec((_D, _D), lambda i: (0, 0)),
            pl.BlockSpec((1, _D), lambda i: (0, 0)),
            pl.BlockSpec((_D, _D), lambda i: (0, 0)),
            pl.BlockSpec((1, _D), lambda i: (0, 0)),
        ],
        out_specs=pl.BlockSpec((_G, _D), lambda i: (0, 0)),
        out_shape=jax.ShapeDtypeStruct((_G, _D), jnp.float32),
    )(agg, parts4, gids2, w2, b2_2, w_out, b_out2, w_ff, b_ff2)


def _pad_edges(e):
    e2 = e.reshape(_NW, _E // _NW)
    npad = _CPW * _CH - _E // _NW
    pad = (_N + (jnp.arange(_NW, dtype=jnp.int32)[:, None] * 7
                 + jnp.arange(npad, dtype=jnp.int32)[None, :]) % (_NP - _N))
    return jnp.concatenate([e2, pad], axis=1).reshape(_NW, _CPW, _CH)


def kernel(x, edge_index, graph_ids, W_in, b_in, W0, b0, W1, b1, W2, b2,
           W_out, b_out, W_ff, b_ff):
    src3 = _pad_edges(edge_index[0].astype(jnp.int32))
    dst3 = _pad_edges(edge_index[1].astype(jnp.int32))
    gids2 = jnp.pad(graph_ids.astype(jnp.int32), (0, _NP - _N),
                    constant_values=_G).reshape(_NP, 1)
    xp = jnp.pad(x, ((0, _NP - _N), (0, 0)))

    deg_parts = _sc_degrees(src3, dst3)
    parts4 = deg_parts.reshape(_NC, 2, _NP, 1)

    h = _tc_embed(xp, W_in, b_in.reshape(1, _D), parts4)
    agg = _sc_conv(h, src3, dst3)
    h = _tc_update(agg, parts4, W0, b0.reshape(1, _D))
    agg = _sc_conv(h, src3, dst3)
    h = _tc_update(agg, parts4, W1, b1.reshape(1, _D))
    agg = _sc_conv(h, src3, dst3)
    return _tc_final(agg, parts4, gids2, W2, b2.reshape(1, _D),
                     W_out, b_out.reshape(1, _D), W_ff, b_ff.reshape(1, _D))


# speedup vs baseline: 1.0038x; 1.0038x over previous
import functools

import jax
import jax.numpy as jnp
from jax import lax
from jax.experimental import pallas as pl
from jax.experimental.pallas import tpu as pltpu
from jax.experimental.pallas import tpu_sc as plsc

_N = 10000
_NP = 10240
_E = 320000
_D = 128
_G = 64
_CH = 128
_NC = 2
_NS = 16
_NW = _NC * _NS
_CPW = 79
_TCB = 1024


def _sc_degrees(src3, dst3):
    mesh = plsc.VectorSubcoreMesh(core_axis_name="c", subcore_axis_name="s")

    @functools.partial(
        pl.kernel,
        mesh=mesh,
        out_type=jax.ShapeDtypeStruct((_NC, 2, _NP), jnp.float32),
        scratch_types=[
            pltpu.VMEM((_CPW, _CH), jnp.int32),
            pltpu.VMEM((_CPW, _CH), jnp.int32),
            pltpu.VMEM((_CH,), jnp.float32),
            pltpu.VMEM((1024,), jnp.float32),
            pltpu.VMEM_SHARED((_NP,), jnp.float32),
            pltpu.VMEM_SHARED((_NP,), jnp.float32),
            pltpu.SemaphoreType.DMA,
            pltpu.SemaphoreType.DMA,
        ],
    )
    def deg_kernel(src_hbm, dst_hbm, out_hbm, idxs2, idxd2, ones_v, zb_v,
                   acc_s, acc_d, sem_s, sem_d):
        c = lax.axis_index("c")
        s = lax.axis_index("s")
        wid = s * _NC + c
        pltpu.sync_copy(src_hbm.at[wid], idxs2)
        pltpu.sync_copy(dst_hbm.at[wid], idxd2)
        for j in range(_CH // 16):
            ones_v[pl.ds(j * 16, 16)] = jnp.ones((16,), jnp.float32)

        @pl.when(s == 0)
        def _zero():
            for j in range(1024 // 16):
                zb_v[pl.ds(j * 16, 16)] = jnp.zeros((16,), jnp.float32)
            for k in range(_NP // 1024):
                pltpu.sync_copy(zb_v, acc_s.at[pl.ds(k * 1024, 1024)])
                pltpu.sync_copy(zb_v, acc_d.at[pl.ds(k * 1024, 1024)])

        plsc.subcore_barrier()

        def body(j, carry):
            pltpu.async_copy(ones_v, acc_s.at[idxs2.at[j]], sem_s, add=True)
            pltpu.async_copy(ones_v, acc_d.at[idxd2.at[j]], sem_d, add=True)
            return carry

        lax.fori_loop(0, _CPW, body, 0)

        def drain(j, carry):
            pltpu.make_async_copy(ones_v, acc_s.at[idxs2.at[0]], sem_s).wait()
            pltpu.make_async_copy(ones_v, acc_d.at[idxd2.at[0]], sem_d).wait()
            return carry

        lax.fori_loop(0, _CPW, drain, 0)
        plsc.subcore_barrier()

        @pl.when(s == 0)
        def _out():
            pltpu.sync_copy(acc_s, out_hbm.at[c, 0])
            pltpu.sync_copy(acc_d, out_hbm.at[c, 1])

    return deg_kernel(src3, dst3)


def _sc_conv(h, src3, dst3):
    mesh = plsc.VectorSubcoreMesh(core_axis_name="c", subcore_axis_name="s")

    @functools.partial(
        pl.kernel,
        mesh=mesh,
        out_type=jax.ShapeDtypeStruct((_NC, _NP, _D), jnp.float32),
        scratch_types=[
            pltpu.VMEM((_CPW, _CH), jnp.int32),
            pltpu.VMEM((_CH,), jnp.int32),
            pltpu.VMEM((_CH,), jnp.int32),
            pltpu.VMEM((_CH, _D), jnp.float32),
            pltpu.VMEM((_CH, _D), jnp.float32),
            pltpu.VMEM_SHARED((_NP, _D), jnp.float32),
            pltpu.SemaphoreType.DMA,
            pltpu.SemaphoreType.DMA,
            pltpu.SemaphoreType.DMA,
            pltpu.SemaphoreType.DMA,
            pltpu.SemaphoreType.DMA,
        ],
    )
    def conv_kernel(h_hbm, src_hbm, dst_hbm, out_hbm, idxs2, idxd_a, idxd_b,
                    rows_a, rows_b, acc, sem_a, sem_b, sem_ia, sem_ib, sem_z):
        c = lax.axis_index("c")
        s = lax.axis_index("s")
        wid = s * _NC + c
        slab = pltpu.async_copy(src_hbm.at[wid], idxs2, sem_z)

        def zrow(i, carry):
            for j in range(_D // 16):
                rows_a[i, pl.ds(j * 16, 16)] = jnp.zeros((16,), jnp.float32)
            return carry

        lax.fori_loop(0, _CH, zrow, 0)
        slab.wait()
        for k in range(5):
            pltpu.async_copy(rows_a, acc.at[pl.ds((s + 16 * k) * _CH, _CH)],
                             sem_z)
        for k in range(5):
            pltpu.make_async_copy(rows_a, acc.at[pl.ds(0, _CH)], sem_z).wait()
        plsc.subcore_barrier()

        def wait_a():
            pltpu.make_async_copy(h_hbm.at[pl.ds(0, _CH)], rows_a, sem_a).wait()
            pltpu.make_async_copy(dst_hbm.at[0, 0], idxd_a, sem_ia).wait()

        def wait_b():
            pltpu.make_async_copy(h_hbm.at[pl.ds(0, _CH)], rows_b, sem_b).wait()
            pltpu.make_async_copy(dst_hbm.at[0, 0], idxd_b, sem_ib).wait()

        pltpu.async_copy(h_hbm.at[idxs2.at[0]], rows_a, sem_a)
        pltpu.async_copy(dst_hbm.at[wid, 0], idxd_a, sem_ia)

        def body(i, carry):
            pltpu.async_copy(h_hbm.at[idxs2.at[2 * i + 1]], rows_b, sem_b)
            pltpu.async_copy(dst_hbm.at[wid, 2 * i + 1], idxd_b, sem_ib)
            wait_a()
            pltpu.sync_copy(rows_a, acc.at[idxd_a], add=True)
            pltpu.async_copy(h_hbm.at[idxs2.at[2 * i + 2]], rows_a, sem_a)
            pltpu.async_copy(dst_hbm.at[wid, 2 * i + 2], idxd_a, sem_ia)
            wait_b()
            pltpu.sync_copy(rows_b, acc.at[idxd_b], add=True)
            return carry

        lax.fori_loop(0, (_CPW - 1) // 2, body, 0)
        wait_a()
        pltpu.sync_copy(rows_a, acc.at[idxd_a], add=True)

        plsc.subcore_barrier()
        for k in range(5):
            off = (s + 16 * k) * _CH
            pltpu.async_copy(acc.at[pl.ds(off, _CH)],
                             out_hbm.at[c, pl.ds(off, _CH)], sem_z)
        for k in range(5):
            pltpu.make_async_copy(acc.at[pl.ds(0, _CH)],
                                  out_hbm.at[c, pl.ds(0, _CH)], sem_z).wait()

    return conv_kernel(h, src3, dst3)


def _norms(p, col):
    deg = p[0, col] + p[1, col]
    return jnp.where(deg > 0, lax.rsqrt(deg), 0.0)


def _tc_embed_raw(x, w, b2):
    def kf(x_ref, w_ref, b_ref, o_ref):
        h = jnp.dot(x_ref[...], w_ref[...],
                    preferred_element_type=jnp.float32) + b_ref[...]
        o_ref[...] = h * jax.nn.sigmoid(h)

    return pl.pallas_call(
        kf,
        grid=(_NP // _TCB,),
        in_specs=[
            pl.BlockSpec((_TCB, _D), lambda i: (i, 0)),
            pl.BlockSpec((_D, _D), lambda i: (0, 0)),
            pl.BlockSpec((1, _D), lambda i: (0, 0)),
        ],
        out_specs=pl.BlockSpec((_TCB, _D), lambda i: (i, 0)),
        out_shape=jax.ShapeDtypeStruct((_NP, _D), jnp.float32),
    )(x, w, b2)


def _tc_scale(h, parts4):
    def kf(h_ref, p_ref, o_ref):
        o_ref[...] = h_ref[...] * _norms(p_ref[...], 0)

    return pl.pallas_call(
        kf,
        grid=(_NP // _TCB,),
        in_specs=[
            pl.BlockSpec((_TCB, _D), lambda i: (i, 0)),
            pl.BlockSpec((2, 2, _TCB, 1), lambda i: (0, 0, i, 0)),
        ],
        out_specs=pl.BlockSpec((_TCB, _D), lambda i: (i, 0)),
        out_shape=jax.ShapeDtypeStruct((_NP, _D), jnp.float32),
    )(h, parts4)


def _tc_update(agg, parts4, w, b2):
    def kf(a_ref, p_ref, w_ref, b_ref, o_ref):
        p = p_ref[...]
        a = (a_ref[0] + a_ref[1]) * _norms(p, 1)
        h = jnp.dot(a, w_ref[...], preferred_element_type=jnp.float32) + b_ref[...]
        h = h * jax.nn.sigmoid(h)
        o_ref[...] = h * _norms(p, 0)

    return pl.pallas_call(
        kf,
        grid=(_NP // _TCB,),
        in_specs=[
            pl.BlockSpec((_NC, _TCB, _D), lambda i: (0, i, 0)),
            pl.BlockSpec((2, 2, _TCB, 1), lambda i: (0, 0, i, 0)),
            pl.BlockSpec((_D, _D), lambda i: (0, 0)),
            pl.BlockSpec((1, _D), lambda i: (0, 0)),
        ],
        out_specs=pl.BlockSpec((_TCB, _D), lambda i: (i, 0)),
        out_shape=jax.ShapeDtypeStruct((_NP, _D), jnp.float32),
    )(agg, parts4, w, b2)


def _tc_final(agg, parts4, gids2, w2, b2_2, w_out, b_out2, w_ff, b_ff2):
    def kf(a_ref, p_ref, g_ref, w2_ref, b2_ref, wo_ref, bo_ref, wf_ref, bf_ref,
           o_ref):
        i = pl.program_id(0)
        a = (a_ref[0] + a_ref[1]) * _norms(p_ref[...], 1)
        h = jnp.dot(a, w2_ref[...], preferred_element_type=jnp.float32) + b2_ref[...]
        h = h * jax.nn.sigmoid(h)
        h = jnp.dot(h, wo_ref[...], preferred_element_type=jnp.float32) + bo_ref[...]
        h = h * jax.nn.sigmoid(h)
        hw = jnp.dot(h, wf_ref[...], preferred_element_type=jnp.float32)
        onehot = (g_ref[...] == lax.broadcasted_iota(jnp.int32, (1, _G), 1))
        part = lax.dot_general(onehot.astype(jnp.float32), hw,
                               (((0,), (0,)), ((), ())),
                               preferred_element_type=jnp.float32)

        @pl.when(i == 0)
        def _init():
            o_ref[...] = jnp.zeros_like(o_ref)

        o_ref[...] += part

        @pl.when(i == pl.num_programs(0) - 1)
        def _bias():
            o_ref[...] += bf_ref[...]

    return pl.pallas_call(
        kf,
        grid=(_NP // _TCB,),
        in_specs=[
            pl.BlockSpec((_NC, _TCB, _D), lambda i: (0, i, 0)),
            pl.BlockSpec((2, 2, _TCB, 1), lambda i: (0, 0, i, 0)),
            pl.BlockSpec((_TCB, 1), lambda i: (i, 0)),
            pl.BlockSpec((_D, _D), lambda i: (0, 0)),
            pl.BlockSpec((1, _D), lambda i: (0, 0)),
            pl.BlockSpec((_D, _D), lambda i: (0, 0)),
            pl.BlockSpec((1, _D), lambda i: (0, 0)),
            pl.BlockSpec((_D, _D), lambda i: (0, 0)),
            pl.BlockSpec((1, _D), lambda i: (0, 0)),
        ],
        out_specs=pl.BlockSpec((_G, _D), lambda i: (0, 0)),
        out_shape=jax.ShapeDtypeStruct((_G, _D), jnp.float32),
    )(agg, parts4, gids2, w2, b2_2, w_out, b_out2, w_ff, b_ff2)


def _pad_edges(e):
    e2 = e.reshape(_NW, _E // _NW)
    npad = _CPW * _CH - _E // _NW
    pad = (_N + (jnp.arange(_NW, dtype=jnp.int32)[:, None] * 7
                 + jnp.arange(npad, dtype=jnp.int32)[None, :]) % (_NP - _N))
    return jnp.concatenate([e2, pad], axis=1).reshape(_NW, _CPW, _CH)


def kernel(x, edge_index, graph_ids, W_in, b_in, W0, b0, W1, b1, W2, b2,
           W_out, b_out, W_ff, b_ff):
    src3 = _pad_edges(edge_index[0].astype(jnp.int32))
    dst3 = _pad_edges(edge_index[1].astype(jnp.int32))
    gids2 = jnp.pad(graph_ids.astype(jnp.int32), (0, _NP - _N),
                    constant_values=_G).reshape(_NP, 1)
    xp = jnp.pad(x, ((0, _NP - _N), (0, 0)))

    h0 = _tc_embed_raw(xp, W_in, b_in.reshape(1, _D))
    deg_parts = _sc_degrees(src3, dst3)
    parts4 = deg_parts.reshape(_NC, 2, _NP, 1)

    h = _tc_scale(h0, parts4)
    agg = _sc_conv(h, src3, dst3)
    h = _tc_update(agg, parts4, W0, b0.reshape(1, _D))
    agg = _sc_conv(h, src3, dst3)
    h = _tc_update(agg, parts4, W1, b1.reshape(1, _D))
    agg = _sc_conv(h, src3, dst3)
    return _tc_final(agg, parts4, gids2, W2, b2.reshape(1, _D),
                     W_out, b_out.reshape(1, _D), W_ff, b_ff.reshape(1, _D))


# speedup vs baseline: 1.0066x; 1.0028x over previous
import functools

import jax
import jax.numpy as jnp
from jax import lax
from jax.experimental import pallas as pl
from jax.experimental.pallas import tpu as pltpu
from jax.experimental.pallas import tpu_sc as plsc

_N = 10000
_NP = 10240
_E = 320000
_D = 128
_G = 64
_CH = 128
_NC = 2
_NS = 16
_NW = _NC * _NS
_CPW = 79
_TCB = 1024


def _sc_degrees(src3, dst3):
    mesh = plsc.VectorSubcoreMesh(core_axis_name="c", subcore_axis_name="s")

    @functools.partial(
        pl.kernel,
        mesh=mesh,
        out_type=jax.ShapeDtypeStruct((_NC, 2, _NP), jnp.float32),
        scratch_types=[
            pltpu.VMEM((_CPW, _CH), jnp.int32),
            pltpu.VMEM((_CPW, _CH), jnp.int32),
            pltpu.VMEM((_CH,), jnp.float32),
            pltpu.VMEM((1024,), jnp.float32),
            pltpu.VMEM_SHARED((_NP,), jnp.float32),
            pltpu.VMEM_SHARED((_NP,), jnp.float32),
            pltpu.SemaphoreType.DMA,
            pltpu.SemaphoreType.DMA,
        ],
    )
    def deg_kernel(src_hbm, dst_hbm, out_hbm, idxs2, idxd2, ones_v, zb_v,
                   acc_s, acc_d, sem_s, sem_d):
        c = lax.axis_index("c")
        s = lax.axis_index("s")
        wid = s * _NC + c
        pltpu.sync_copy(src_hbm.at[wid], idxs2)
        pltpu.sync_copy(dst_hbm.at[wid], idxd2)
        for j in range(_CH // 16):
            ones_v[pl.ds(j * 16, 16)] = jnp.ones((16,), jnp.float32)

        @pl.when(s == 0)
        def _zero():
            for j in range(1024 // 16):
                zb_v[pl.ds(j * 16, 16)] = jnp.zeros((16,), jnp.float32)
            for k in range(_NP // 1024):
                pltpu.sync_copy(zb_v, acc_s.at[pl.ds(k * 1024, 1024)])
                pltpu.sync_copy(zb_v, acc_d.at[pl.ds(k * 1024, 1024)])

        plsc.subcore_barrier()

        def body(j, carry):
            pltpu.async_copy(ones_v, acc_s.at[idxs2.at[j]], sem_s, add=True)
            pltpu.async_copy(ones_v, acc_d.at[idxd2.at[j]], sem_d, add=True)
            return carry

        lax.fori_loop(0, _CPW, body, 0)

        def drain(j, carry):
            pltpu.make_async_copy(ones_v, acc_s.at[idxs2.at[0]], sem_s).wait()
            pltpu.make_async_copy(ones_v, acc_d.at[idxd2.at[0]], sem_d).wait()
            return carry

        lax.fori_loop(0, _CPW, drain, 0)
        plsc.subcore_barrier()

        @pl.when(s == 0)
        def _out():
            pltpu.sync_copy(acc_s, out_hbm.at[c, 0])
            pltpu.sync_copy(acc_d, out_hbm.at[c, 1])

    return deg_kernel(src3, dst3)


def _sc_conv(h, src3, dst3):
    mesh = plsc.VectorSubcoreMesh(core_axis_name="c", subcore_axis_name="s")

    @functools.partial(
        pl.kernel,
        mesh=mesh,
        out_type=jax.ShapeDtypeStruct((_NC, _NP, _D), jnp.float32),
        scratch_types=[
            pltpu.VMEM((_CPW, _CH), jnp.int32),
            pltpu.VMEM((_CH,), jnp.int32),
            pltpu.VMEM((_CH,), jnp.int32),
            pltpu.VMEM((_CH, _D), jnp.float32),
            pltpu.VMEM((_CH, _D), jnp.float32),
            pltpu.VMEM_SHARED((_NP, _D), jnp.float32),
            pltpu.SemaphoreType.DMA,
            pltpu.SemaphoreType.DMA,
            pltpu.SemaphoreType.DMA,
            pltpu.SemaphoreType.DMA,
            pltpu.SemaphoreType.DMA,
        ],
    )
    def conv_kernel(h_hbm, src_hbm, dst_hbm, out_hbm, idxs2, idxd_a, idxd_b,
                    rows_a, rows_b, acc, sem_a, sem_b, sem_ia, sem_ib, sem_z):
        c = lax.axis_index("c")
        s = lax.axis_index("s")
        wid = s * _NC + c
        slab = pltpu.async_copy(src_hbm.at[wid], idxs2, sem_z)

        def zrow(i, carry):
            for j in range(_D // 16):
                rows_a[i, pl.ds(j * 16, 16)] = jnp.zeros((16,), jnp.float32)
            return carry

        lax.fori_loop(0, _CH, zrow, 0)
        slab.wait()
        for k in range(5):
            pltpu.async_copy(rows_a, acc.at[pl.ds((s + 16 * k) * _CH, _CH)],
                             sem_z)
        for k in range(5):
            pltpu.make_async_copy(rows_a, acc.at[pl.ds(0, _CH)], sem_z).wait()
        plsc.subcore_barrier()

        def wait_a():
            pltpu.make_async_copy(h_hbm.at[pl.ds(0, _CH)], rows_a, sem_a).wait()
            pltpu.make_async_copy(dst_hbm.at[0, 0], idxd_a, sem_ia).wait()

        def wait_b():
            pltpu.make_async_copy(h_hbm.at[pl.ds(0, _CH)], rows_b, sem_b).wait()
            pltpu.make_async_copy(dst_hbm.at[0, 0], idxd_b, sem_ib).wait()

        pltpu.async_copy(h_hbm.at[idxs2.at[0]], rows_a, sem_a)
        pltpu.async_copy(dst_hbm.at[wid, 0], idxd_a, sem_ia)

        def body(i, carry):
            pltpu.async_copy(h_hbm.at[idxs2.at[2 * i + 1]], rows_b, sem_b)
            pltpu.async_copy(dst_hbm.at[wid, 2 * i + 1], idxd_b, sem_ib)
            wait_a()
            pltpu.sync_copy(rows_a, acc.at[idxd_a], add=True)
            pltpu.async_copy(h_hbm.at[idxs2.at[2 * i + 2]], rows_a, sem_a)
            pltpu.async_copy(dst_hbm.at[wid, 2 * i + 2], idxd_a, sem_ia)
            wait_b()
            pltpu.sync_copy(rows_b, acc.at[idxd_b], add=True)
            return carry

        lax.fori_loop(0, (_CPW - 1) // 2, body, 0)
        wait_a()
        pltpu.sync_copy(rows_a, acc.at[idxd_a], add=True)

        plsc.subcore_barrier()
        for k in range(5):
            off = (s + 16 * k) * _CH
            pltpu.sync_copy(acc.at[pl.ds(off, _CH)],
                            out_hbm.at[c, pl.ds(off, _CH)])

    return conv_kernel(h, src3, dst3)


def _norms(p, col):
    deg = p[0, col] + p[1, col]
    return jnp.where(deg > 0, lax.rsqrt(deg), 0.0)


def _tc_embed_raw(x, w, b2):
    def kf(x_ref, w_ref, b_ref, o_ref):
        h = jnp.dot(x_ref[...], w_ref[...],
                    preferred_element_type=jnp.float32) + b_ref[...]
        o_ref[...] = h * jax.nn.sigmoid(h)

    return pl.pallas_call(
        kf,
        grid=(_NP // _TCB,),
        in_specs=[
            pl.BlockSpec((_TCB, _D), lambda i: (i, 0)),
            pl.BlockSpec((_D, _D), lambda i: (0, 0)),
            pl.BlockSpec((1, _D), lambda i: (0, 0)),
        ],
        out_specs=pl.BlockSpec((_TCB, _D), lambda i: (i, 0)),
        out_shape=jax.ShapeDtypeStruct((_NP, _D), jnp.float32),
    )(x, w, b2)


def _tc_scale(h, parts4):
    def kf(h_ref, p_ref, o_ref):
        o_ref[...] = h_ref[...] * _norms(p_ref[...], 0)

    return pl.pallas_call(
        kf,
        grid=(_NP // _TCB,),
        in_specs=[
            pl.BlockSpec((_TCB, _D), lambda i: (i, 0)),
            pl.BlockSpec((2, 2, _TCB, 1), lambda i: (0, 0, i, 0)),
        ],
        out_specs=pl.BlockSpec((_TCB, _D), lambda i: (i, 0)),
        out_shape=jax.ShapeDtypeStruct((_NP, _D), jnp.float32),
    )(h, parts4)


def _tc_update(agg, parts4, w, b2):
    def kf(a_ref, p_ref, w_ref, b_ref, o_ref):
        p = p_ref[...]
        a = (a_ref[0] + a_ref[1]) * _norms(p, 1)
        h = jnp.dot(a, w_ref[...], preferred_element_type=jnp.float32) + b_ref[...]
        h = h * jax.nn.sigmoid(h)
        o_ref[...] = h * _norms(p, 0)

    return pl.pallas_call(
        kf,
        grid=(_NP // _TCB,),
        in_specs=[
            pl.BlockSpec((_NC, _TCB, _D), lambda i: (0, i, 0)),
            pl.BlockSpec((2, 2, _TCB, 1), lambda i: (0, 0, i, 0)),
            pl.BlockSpec((_D, _D), lambda i: (0, 0)),
            pl.BlockSpec((1, _D), lambda i: (0, 0)),
        ],
        out_specs=pl.BlockSpec((_TCB, _D), lambda i: (i, 0)),
        out_shape=jax.ShapeDtypeStruct((_NP, _D), jnp.float32),
    )(agg, parts4, w, b2)


def _tc_final(agg, parts4, gids2, w2, b2_2, w_out, b_out2, w_ff, b_ff2):
    def kf(a_ref, p_ref, g_ref, w2_ref, b2_ref, wo_ref, bo_ref, wf_ref, bf_ref,
           o_ref):
        i = pl.program_id(0)
        a = (a_ref[0] + a_ref[1]) * _norms(p_ref[...], 1)
        h = jnp.dot(a, w2_ref[...], preferred_element_type=jnp.float32) + b2_ref[...]
        h = h * jax.nn.sigmoid(h)
        h = jnp.dot(h, wo_ref[...], preferred_element_type=jnp.float32) + bo_ref[...]
        h = h * jax.nn.sigmoid(h)
        hw = jnp.dot(h, wf_ref[...], preferred_element_type=jnp.float32)
        onehot = (g_ref[...] == lax.broadcasted_iota(jnp.int32, (1, _G), 1))
        part = lax.dot_general(onehot.astype(jnp.float32), hw,
                               (((0,), (0,)), ((), ())),
                               preferred_element_type=jnp.float32)

        @pl.when(i == 0)
        def _init():
            o_ref[...] = jnp.zeros_like(o_ref)

        o_ref[...] += part

        @pl.when(i == pl.num_programs(0) - 1)
        def _bias():
            o_ref[...] += bf_ref[...]

    return pl.pallas_call(
        kf,
        grid=(_NP // _TCB,),
        in_specs=[
            pl.BlockSpec((_NC, _TCB, _D), lambda i: (0, i, 0)),
            pl.BlockSpec((2, 2, _TCB, 1), lambda i: (0, 0, i, 0)),
            pl.BlockSpec((_TCB, 1), lambda i: (i, 0)),
            pl.BlockSpec((_D, _D), lambda i: (0, 0)),
            pl.BlockSpec((1, _D), lambda i: (0, 0)),
            pl.BlockSpec((_D, _D), lambda i: (0, 0)),
            pl.BlockSpec((1, _D), lambda i: (0, 0)),
            pl.BlockSpec((_D, _D), lambda i: (0, 0)),
            pl.BlockSpec((1, _D), lambda i: (0, 0)),
        ],
        out_specs=pl.BlockSpec((_G, _D), lambda i: (0, 0)),
        out_shape=jax.ShapeDtypeStruct((_G, _D), jnp.float32),
    )(agg, parts4, gids2, w2, b2_2, w_out, b_out2, w_ff, b_ff2)


def _pad_edges(e):
    e2 = e.reshape(_NW, _E // _NW)
    npad = _CPW * _CH - _E // _NW
    pad = (_N + (jnp.arange(_NW, dtype=jnp.int32)[:, None] * 7
                 + jnp.arange(npad, dtype=jnp.int32)[None, :]) % (_NP - _N))
    return jnp.concatenate([e2, pad], axis=1).reshape(_NW, _CPW, _CH)


def kernel(x, edge_index, graph_ids, W_in, b_in, W0, b0, W1, b1, W2, b2,
           W_out, b_out, W_ff, b_ff):
    src3 = _pad_edges(edge_index[0].astype(jnp.int32))
    dst3 = _pad_edges(edge_index[1].astype(jnp.int32))
    gids2 = jnp.pad(graph_ids.astype(jnp.int32), (0, _NP - _N),
                    constant_values=_G).reshape(_NP, 1)
    xp = jnp.pad(x, ((0, _NP - _N), (0, 0)))

    h0 = _tc_embed_raw(xp, W_in, b_in.reshape(1, _D))
    deg_parts = _sc_degrees(src3, dst3)
    parts4 = deg_parts.reshape(_NC, 2, _NP, 1)

    h = _tc_scale(h0, parts4)
    agg = _sc_conv(h, src3, dst3)
    h = _tc_update(agg, parts4, W0, b0.reshape(1, _D))
    agg = _sc_conv(h, src3, dst3)
    h = _tc_update(agg, parts4, W1, b1.reshape(1, _D))
    agg = _sc_conv(h, src3, dst3)
    return _tc_final(agg, parts4, gids2, W2, b2.reshape(1, _D),
                     W_out, b_out.reshape(1, _D), W_ff, b_ff.reshape(1, _D))
